# baseline (speedup 1.0000x reference)
import functools

import jax
import jax.numpy as jnp
from jax import lax
from jax.experimental import pallas as pl
from jax.experimental.pallas import tpu as pltpu
from jax.experimental.pallas import tpu_sc as plsc

B = 1024
OBS_D = 512
E = 128
DB = 100000
K = 4

BT = 1024
DT = 4096
NBT = B // BT
NDT = (DB + DT - 1) // DT

_NC = 2
_NS = 16
_NW = _NC * _NS
_BPW = (B * K) // _NW


def _topk_body(obs_ref, w_ref, db_ref, idx_ref, q_s, rv_s, ri_s, cm_s, p_s):
    dt = pl.program_id(1)

    @pl.when(dt == 0)
    def _init():
        q_s[...] = jnp.dot(
            obs_ref[...], w_ref[...],
            preferred_element_type=jnp.float32)
        rv_s[...] = jnp.full(rv_s.shape, -jnp.inf, jnp.float32)
        ri_s[...] = jnp.full(ri_s.shape, jnp.inf, jnp.float32)
        cm_s[...] = jnp.zeros((DT, 128), jnp.float32)
        ir = lax.broadcasted_iota(jnp.int32, (DT, 1), 0)
        cm_s[:, 0:1] = (ir >> 6).astype(jnp.float32)
        cm_s[:, 1:2] = (ir & 63).astype(jnp.float32)
        cm_s[:, 2:3] = jnp.ones((DT, 1), jnp.float32)

    sim = lax.dot_general(
        q_s[...], db_ref[...],
        (((1,), (1,)), ((), ())),
        preferred_element_type=jnp.float32)
    col = lax.broadcasted_iota(jnp.int32, (BT, DT), 1).astype(jnp.float32)
    thr = (DB - dt * DT).astype(jnp.float32)
    sim = jnp.where(col < thr, sim, -jnp.inf)

    off = (dt * DT).astype(jnp.float32)
    tv, ti = [], []
    for j in range(K):
        m = jnp.max(sim, axis=1, keepdims=True)
        eqf = jnp.where(sim == m, 1.0, 0.0)
        r = lax.dot_general(
            eqf, cm_s[...], (((1,), (0,)), ((), ())),
            preferred_element_type=jnp.float32)
        p_s[...] = r[:, 0:1] * 64.0 + r[:, 1:2]
        cur = sim

        @pl.when(jnp.max(r[:, 2:3]) > 1.0)
        def _dup_fallback():
            p_s[...] = jnp.min(jnp.where(cur == m, col, jnp.inf),
                               axis=1, keepdims=True)

        p = p_s[...]
        tv.append(m)
        ti.append(p + off)
        if j < K - 1:
            sim = jnp.where(col == p, -jnp.inf, sim)

    cvals = jnp.concatenate([rv_s[...]] + tv, axis=1)
    cidx = jnp.concatenate([ri_s[...]] + ti, axis=1)
    nv, ni = [], []
    for j in range(K):
        m = jnp.max(cvals, axis=1, keepdims=True)
        s = jnp.min(jnp.where(cvals == m, cidx, jnp.inf),
                    axis=1, keepdims=True)
        nv.append(m)
        ni.append(s)
        if j < K - 1:
            cvals = jnp.where(cidx == s, -jnp.inf, cvals)
    rv_s[...] = jnp.concatenate(nv, axis=1)
    ri_s[...] = jnp.concatenate(ni, axis=1)

    @pl.when(dt == NDT - 1)
    def _finish():
        idx_ref[...] = ri_s[...].astype(jnp.int32)


def _topk_call(obs, w, db):
    return pl.pallas_call(
        _topk_body,
        grid=(NBT, NDT),
        in_specs=[
            pl.BlockSpec((BT, OBS_D), lambda bt, dt: (bt, 0)),
            pl.BlockSpec((OBS_D, E), lambda bt, dt: (0, 0)),
            pl.BlockSpec((DT, E), lambda bt, dt: (dt, 0)),
        ],
        out_specs=pl.BlockSpec((BT, K), lambda bt, dt: (bt, 0)),
        out_shape=jax.ShapeDtypeStruct((B, K), jnp.int32),
        scratch_shapes=[
            pltpu.VMEM((BT, E), jnp.float32),
            pltpu.VMEM((BT, K), jnp.float32),
            pltpu.VMEM((BT, K), jnp.float32),
            pltpu.VMEM((DT, 128), jnp.float32),
            pltpu.VMEM((BT, 1), jnp.float32),
        ],
        compiler_params=pltpu.CompilerParams(
            dimension_semantics=("arbitrary", "arbitrary")),
    )(obs, w, db)


def _gather_body(db_hbm, idx_hbm, out_hbm, idx_v, rows_v, sem):
    wid = lax.axis_index("s") * _NC + lax.axis_index("c")
    base = wid * _BPW
    pltpu.sync_copy(idx_hbm.at[pl.ds(base, _BPW)], idx_v)
    pltpu.async_copy(db_hbm.at[idx_v], rows_v, sem).wait()
    pltpu.sync_copy(rows_v, out_hbm.at[pl.ds(base, _BPW)])


@functools.lru_cache(maxsize=1)
def _gather_call():
    return pl.kernel(
        _gather_body,
        mesh=plsc.VectorSubcoreMesh(core_axis_name="c", subcore_axis_name="s"),
        out_type=jax.ShapeDtypeStruct((B * K, E), jnp.float32),
        scratch_types=[
            pltpu.VMEM((_BPW,), jnp.int32),
            pltpu.VMEM((_BPW, E), jnp.float32),
            pltpu.SemaphoreType.DMA,
        ],
    )


def kernel(obs, W_obs, db_embeddings, top_k):
    del top_k
    idx = _topk_call(obs, W_obs, db_embeddings)
    rows = _gather_call()(db_embeddings, idx.reshape(B * K))
    memory = rows.reshape(B, K * E)
    return memory, idx


# speedup vs baseline: 1.7266x; 1.7266x over previous
import functools

import jax
import jax.numpy as jnp
from jax import lax
from jax.experimental import pallas as pl
from jax.experimental.pallas import tpu as pltpu
from jax.experimental.pallas import tpu_sc as plsc

B = 1024
OBS_D = 512
E = 128
DB = 100000
K = 4

BT = 1024
DT = 8192
NBT = B // BT
NDT = (DB + DT - 1) // DT

_NC = 2
_NS = 16
_NW = _NC * _NS
_BPW = (B * K) // _NW


def _topk_body(obs_ref, w_ref, db_ref, idx_ref, q_s, rv_s, ri_s):
    dt = pl.program_id(1)

    @pl.when(dt == 0)
    def _init():
        q_s[...] = jnp.dot(
            obs_ref[...], w_ref[...],
            preferred_element_type=jnp.float32)
        rv_s[...] = jnp.full(rv_s.shape, -jnp.inf, jnp.float32)
        ri_s[...] = jnp.full(ri_s.shape, jnp.inf, jnp.float32)

    sim = lax.dot_general(
        q_s[...], db_ref[...],
        (((1,), (1,)), ((), ())),
        preferred_element_type=jnp.float32)
    col = lax.broadcasted_iota(jnp.int32, (BT, DT), 1).astype(jnp.float32)
    thr = (DB - dt * DT).astype(jnp.float32)
    sim = jnp.where(col < thr, sim, -jnp.inf)

    off = (dt * DT).astype(jnp.float32)
    tv, ti = [], []
    for j in range(K):
        m = jnp.max(sim, axis=1, keepdims=True)
        p = jnp.min(jnp.where(sim == m, col, jnp.inf), axis=1, keepdims=True)
        tv.append(m)
        ti.append(p + off)
        if j < K - 1:
            sim = jnp.where(col == p, -jnp.inf, sim)

    cvals = jnp.concatenate([rv_s[...]] + tv, axis=1)
    cidx = jnp.concatenate([ri_s[...]] + ti, axis=1)
    nv, ni = [], []
    for j in range(K):
        m = jnp.max(cvals, axis=1, keepdims=True)
        s = jnp.min(jnp.where(cvals == m, cidx, jnp.inf),
                    axis=1, keepdims=True)
        nv.append(m)
        ni.append(s)
        if j < K - 1:
            cvals = jnp.where(cidx == s, -jnp.inf, cvals)
    rv_s[...] = jnp.concatenate(nv, axis=1)
    ri_s[...] = jnp.concatenate(ni, axis=1)

    @pl.when(dt == NDT - 1)
    def _finish():
        idx_ref[...] = ri_s[...].astype(jnp.int32)


def _topk_call(obs, w, db):
    return pl.pallas_call(
        _topk_body,
        grid=(NBT, NDT),
        in_specs=[
            pl.BlockSpec((BT, OBS_D), lambda bt, dt: (bt, 0)),
            pl.BlockSpec((OBS_D, E), lambda bt, dt: (0, 0)),
            pl.BlockSpec((DT, E), lambda bt, dt: (dt, 0)),
        ],
        out_specs=pl.BlockSpec((BT, K), lambda bt, dt: (bt, 0)),
        out_shape=jax.ShapeDtypeStruct((B, K), jnp.int32),
        scratch_shapes=[
            pltpu.VMEM((BT, E), jnp.float32),
            pltpu.VMEM((BT, K), jnp.float32),
            pltpu.VMEM((BT, K), jnp.float32),
        ],
        compiler_params=pltpu.CompilerParams(
            dimension_semantics=("arbitrary", "arbitrary")),
    )(obs, w, db)


def _gather_body(db_hbm, idx_hbm, out_hbm, idx_v, rows_v, sem):
    wid = lax.axis_index("s") * _NC + lax.axis_index("c")
    base = wid * _BPW
    pltpu.sync_copy(idx_hbm.at[pl.ds(base, _BPW)], idx_v)
    pltpu.async_copy(db_hbm.at[idx_v], rows_v, sem).wait()
    pltpu.sync_copy(rows_v, out_hbm.at[pl.ds(base, _BPW)])


@functools.lru_cache(maxsize=1)
def _gather_call():
    return pl.kernel(
        _gather_body,
        mesh=plsc.VectorSubcoreMesh(core_axis_name="c", subcore_axis_name="s"),
        out_type=jax.ShapeDtypeStruct((B * K, E), jnp.float32),
        scratch_types=[
            pltpu.VMEM((_BPW,), jnp.int32),
            pltpu.VMEM((_BPW, E), jnp.float32),
            pltpu.SemaphoreType.DMA,
        ],
    )


def kernel(obs, W_obs, db_embeddings, top_k):
    del top_k
    idx = _topk_call(obs, W_obs, db_embeddings)
    rows = _gather_call()(db_embeddings, idx.reshape(B * K))
    memory = rows.reshape(B, K * E)
    return memory, idx


# speedup vs baseline: 1.7490x; 1.0130x over previous
import functools

import jax
import jax.numpy as jnp
from jax import lax
from jax.experimental import pallas as pl
from jax.experimental.pallas import tpu as pltpu
from jax.experimental.pallas import tpu_sc as plsc

B = 1024
OBS_D = 512
E = 128
DB = 100000
K = 4

BT = 1024
DT = 4096
NBT = B // BT
NDT = (DB + DT - 1) // DT

_NC = 2
_NS = 16
_NW = _NC * _NS
_BPW = (B * K) // _NW


def _topk_body(obs_ref, w_ref, db_ref, idx_ref, q_s, rv_s, ri_s):
    dt = pl.program_id(1)

    @pl.when(dt == 0)
    def _init():
        q_s[...] = jnp.dot(
            obs_ref[...], w_ref[...],
            preferred_element_type=jnp.float32)
        rv_s[...] = jnp.full(rv_s.shape, -jnp.inf, jnp.float32)
        ri_s[...] = jnp.full(ri_s.shape, jnp.inf, jnp.float32)

    sim = lax.dot_general(
        q_s[...], db_ref[...],
        (((1,), (1,)), ((), ())),
        preferred_element_type=jnp.float32)
    col = lax.broadcasted_iota(jnp.int32, (BT, DT), 1).astype(jnp.float32)
    thr = (DB - dt * DT).astype(jnp.float32)
    sim = jnp.where(col < thr, sim, -jnp.inf)

    off = (dt * DT).astype(jnp.float32)
    tv, ti = [], []
    for j in range(K):
        m = jnp.max(sim, axis=1, keepdims=True)
        p = jnp.min(jnp.where(sim == m, col, jnp.inf), axis=1, keepdims=True)
        tv.append(m)
        ti.append(p + off)
        if j < K - 1:
            sim = jnp.where(col == p, -jnp.inf, sim)

    cvals = jnp.concatenate([rv_s[...]] + tv, axis=1)
    cidx = jnp.concatenate([ri_s[...]] + ti, axis=1)
    nv, ni = [], []
    for j in range(K):
        m = jnp.max(cvals, axis=1, keepdims=True)
        s = jnp.min(jnp.where(cvals == m, cidx, jnp.inf),
                    axis=1, keepdims=True)
        nv.append(m)
        ni.append(s)
        if j < K - 1:
            cvals = jnp.where(cidx == s, -jnp.inf, cvals)
    rv_s[...] = jnp.concatenate(nv, axis=1)
    ri_s[...] = jnp.concatenate(ni, axis=1)

    @pl.when(dt == NDT - 1)
    def _finish():
        idx_ref[...] = ri_s[...].astype(jnp.int32)


def _topk_call(obs, w, db):
    return pl.pallas_call(
        _topk_body,
        grid=(NBT, NDT),
        in_specs=[
            pl.BlockSpec((BT, OBS_D), lambda bt, dt: (bt, 0)),
            pl.BlockSpec((OBS_D, E), lambda bt, dt: (0, 0)),
            pl.BlockSpec((DT, E), lambda bt, dt: (dt, 0)),
        ],
        out_specs=pl.BlockSpec((BT, K), lambda bt, dt: (bt, 0)),
        out_shape=jax.ShapeDtypeStruct((B, K), jnp.int32),
        scratch_shapes=[
            pltpu.VMEM((BT, E), jnp.float32),
            pltpu.VMEM((BT, K), jnp.float32),
            pltpu.VMEM((BT, K), jnp.float32),
        ],
        compiler_params=pltpu.CompilerParams(
            dimension_semantics=("arbitrary", "arbitrary")),
    )(obs, w, db)


def _gather_body(db_hbm, idx_hbm, out_hbm, idx_v, rows_v, sem):
    wid = lax.axis_index("s") * _NC + lax.axis_index("c")
    base = wid * _BPW
    pltpu.sync_copy(idx_hbm.at[pl.ds(base, _BPW)], idx_v)
    pltpu.async_copy(db_hbm.at[idx_v], rows_v, sem).wait()
    pltpu.sync_copy(rows_v, out_hbm.at[pl.ds(base, _BPW)])


@functools.lru_cache(maxsize=1)
def _gather_call():
    return pl.kernel(
        _gather_body,
        mesh=plsc.VectorSubcoreMesh(core_axis_name="c", subcore_axis_name="s"),
        out_type=jax.ShapeDtypeStruct((B * K, E), jnp.float32),
        scratch_types=[
            pltpu.VMEM((_BPW,), jnp.int32),
            pltpu.VMEM((_BPW, E), jnp.float32),
            pltpu.SemaphoreType.DMA,
        ],
    )


def kernel(obs, W_obs, db_embeddings, top_k):
    del top_k
    idx = _topk_call(obs, W_obs, db_embeddings)
    rows = _gather_call()(db_embeddings, idx.reshape(B * K))
    memory = rows.reshape(B, K * E)
    return memory, idx


# speedup vs baseline: 1.7662x; 1.0098x over previous
import functools

import jax
import jax.numpy as jnp
from jax import lax
from jax.experimental import pallas as pl
from jax.experimental.pallas import tpu as pltpu
from jax.experimental.pallas import tpu_sc as plsc

B = 1024
OBS_D = 512
E = 128
DB = 100000
K = 4

BT = 1024
DT = 4096
NBT = B // BT
NDT = (DB + DT - 1) // DT

_NC = 2
_NS = 16
_NW = _NC * _NS
_BPW = (B * K) // _NW


def _topk_body(obs_ref, w_ref, db_ref, idx_ref, q_s, rv_s, ri_s):
    dt = pl.program_id(1)

    @pl.when(dt == 0)
    def _init():
        q_s[...] = jnp.dot(
            obs_ref[...], w_ref[...],
            preferred_element_type=jnp.float32)
        rv_s[...] = jnp.full(rv_s.shape, -jnp.inf, jnp.float32)
        ri_s[...] = jnp.full(ri_s.shape, jnp.inf, jnp.float32)

    sim = lax.dot_general(
        q_s[...], db_ref[...],
        (((1,), (1,)), ((), ())),
        preferred_element_type=jnp.float32)
    H = DT // 2
    colh = lax.broadcasted_iota(jnp.int32, (BT, H), 1).astype(jnp.float32)
    colh2 = colh + float(H)
    thr = (DB - dt * DT).astype(jnp.float32)
    a = jnp.where(colh < thr, sim[:, :H], -jnp.inf)
    b = jnp.where(colh2 < thr, sim[:, H:], -jnp.inf)
    ge = a >= b
    w = jnp.where(ge, a, b)
    l = jnp.where(ge, b, a)
    wi = jnp.where(ge, colh, colh2)
    li = jnp.where(ge, colh2, colh)

    off = (dt * DT).astype(jnp.float32)
    tv, ti = [], []
    for j in range(K):
        m = jnp.max(w, axis=1, keepdims=True)
        p = jnp.min(jnp.where(w == m, wi, jnp.inf), axis=1, keepdims=True)
        tv.append(m)
        ti.append(p + off)
        if j < K - 1:
            hit = wi == p
            w = jnp.where(hit, l, w)
            wi = jnp.where(hit, li, wi)
            l = jnp.where(hit, -jnp.inf, l)

    cvals = jnp.concatenate([rv_s[...]] + tv, axis=1)
    cidx = jnp.concatenate([ri_s[...]] + ti, axis=1)
    nv, ni = [], []
    for j in range(K):
        m = jnp.max(cvals, axis=1, keepdims=True)
        s = jnp.min(jnp.where(cvals == m, cidx, jnp.inf),
                    axis=1, keepdims=True)
        nv.append(m)
        ni.append(s)
        if j < K - 1:
            cvals = jnp.where(cidx == s, -jnp.inf, cvals)
    rv_s[...] = jnp.concatenate(nv, axis=1)
    ri_s[...] = jnp.concatenate(ni, axis=1)

    @pl.when(dt == NDT - 1)
    def _finish():
        idx_ref[...] = ri_s[...].astype(jnp.int32)


def _topk_call(obs, w, db):
    return pl.pallas_call(
        _topk_body,
        grid=(NBT, NDT),
        in_specs=[
            pl.BlockSpec((BT, OBS_D), lambda bt, dt: (bt, 0)),
            pl.BlockSpec((OBS_D, E), lambda bt, dt: (0, 0)),
            pl.BlockSpec((DT, E), lambda bt, dt: (dt, 0)),
        ],
        out_specs=pl.BlockSpec((BT, K), lambda bt, dt: (bt, 0)),
        out_shape=jax.ShapeDtypeStruct((B, K), jnp.int32),
        scratch_shapes=[
            pltpu.VMEM((BT, E), jnp.float32),
            pltpu.VMEM((BT, K), jnp.float32),
            pltpu.VMEM((BT, K), jnp.float32),
        ],
        compiler_params=pltpu.CompilerParams(
            dimension_semantics=("arbitrary", "arbitrary")),
    )(obs, w, db)


def _gather_body(db_hbm, idx_hbm, out_hbm, idx_v, rows_v, sem):
    wid = lax.axis_index("s") * _NC + lax.axis_index("c")
    base = wid * _BPW
    pltpu.sync_copy(idx_hbm.at[pl.ds(base, _BPW)], idx_v)
    pltpu.async_copy(db_hbm.at[idx_v], rows_v, sem).wait()
    pltpu.sync_copy(rows_v, out_hbm.at[pl.ds(base, _BPW)])


@functools.lru_cache(maxsize=1)
def _gather_call():
    return pl.kernel(
        _gather_body,
        mesh=plsc.VectorSubcoreMesh(core_axis_name="c", subcore_axis_name="s"),
        out_type=jax.ShapeDtypeStruct((B * K, E), jnp.float32),
        scratch_types=[
            pltpu.VMEM((_BPW,), jnp.int32),
            pltpu.VMEM((_BPW, E), jnp.float32),
            pltpu.SemaphoreType.DMA,
        ],
    )


def kernel(obs, W_obs, db_embeddings, top_k):
    del top_k
    idx = _topk_call(obs, W_obs, db_embeddings)
    rows = _gather_call()(db_embeddings, idx.reshape(B * K))
    memory = rows.reshape(B, K * E)
    return memory, idx
